# baseline (speedup 1.0000x reference)
import functools

import jax
import jax.numpy as jnp
from jax import lax
from jax.experimental import pallas as pl
from jax.experimental.pallas import tpu as pltpu
from jax.experimental.pallas import tpu_sc as plsc

NUM_GROUP = 26
NUM_ENV = 100000
HIDDEN_DIM = 32
BATCH = 16384

_INFO = plsc.get_sparse_core_info()
_NC, _NS, _L = _INFO.num_cores, _INFO.num_subcores, _INFO.num_lanes
_NW = _NC * _NS
_BPW = BATCH // _NW
_CHUNK = 64
_NCHUNK = _BPW // _CHUNK
_IDX_PER_CHUNK = _CHUNK * NUM_GROUP
_ISTRIDE = 128
_NSTREAM = _IDX_PER_CHUNK // _ISTRIDE


def _sc_body(idx_hbm, table_hbm, out_hbm, idx_v, rows_v, out_v, sem):
    wid = lax.axis_index("s") * _NC + lax.axis_index("c")
    for chunk in range(_NCHUNK):
        base = wid * _BPW + chunk * _CHUNK
        cid = wid * _NCHUNK + chunk
        pltpu.sync_copy(idx_hbm.at[cid], idx_v)

        def fire(k, _):
            vec = idx_v[pl.ds(k * _L, _L)]
            pltpu.async_copy(
                table_hbm.at[vec], rows_v.at[pl.ds(k * _L, _L)], sem)
            return 0

        lax.fori_loop(0, _IDX_PER_CHUNK // _L, fire, 0)

        def drain(k, _):
            pltpu.make_async_copy(
                table_hbm.at[pl.ds(0, _L)],
                rows_v.at[pl.ds(k * _L, _L)], sem).wait()
            return 0

        lax.fori_loop(0, _IDX_PER_CHUNK // _L, drain, 0)

        def body(i, _):
            r = i * NUM_GROUP
            a0 = rows_v[r, pl.ds(0, _L)]
            a1 = rows_v[r, pl.ds(_L, _L)]
            for g in range(1, NUM_GROUP):
                a0 = a0 + rows_v[r + g, pl.ds(0, _L)]
                a1 = a1 + rows_v[r + g, pl.ds(_L, _L)]
            out_v[i, pl.ds(0, _L)] = a0
            out_v[i, pl.ds(_L, _L)] = a1
            return 0

        lax.fori_loop(0, _CHUNK, body, 0)
        pltpu.sync_copy(out_v, out_hbm.at[pl.ds(base, _CHUNK)])


def _lookup_sum(idx_flat, table_flat):
    mesh = plsc.VectorSubcoreMesh(core_axis_name="c", subcore_axis_name="s")
    kern = functools.partial(
        pl.kernel,
        mesh=mesh,
        out_type=jax.ShapeDtypeStruct((BATCH, HIDDEN_DIM), jnp.float32),
        scratch_types=[
            pltpu.VMEM((_IDX_PER_CHUNK,), jnp.int32),
            pltpu.VMEM((_IDX_PER_CHUNK, HIDDEN_DIM), jnp.float32),
            pltpu.VMEM((_CHUNK, HIDDEN_DIM), jnp.float32),
            pltpu.SemaphoreType.DMA,
        ],
        compiler_params=pltpu.CompilerParams(use_tc_tiling_on_sc=False),
    )(_sc_body)
    return kern(idx_flat, table_flat)


def kernel(envs, tables):
    offs = jnp.arange(NUM_GROUP, dtype=jnp.int32) * NUM_ENV
    idx_flat = (envs.astype(jnp.int32) + offs[None, :]).reshape(
        _NW * _NCHUNK, _IDX_PER_CHUNK)
    table_flat = tables.reshape(NUM_GROUP * NUM_ENV, HIDDEN_DIM)
    return _lookup_sum(idx_flat, table_flat)


# speedup vs baseline: 1.0055x; 1.0055x over previous
import functools

import jax
import jax.numpy as jnp
from jax import lax
from jax.experimental import pallas as pl
from jax.experimental.pallas import tpu as pltpu
from jax.experimental.pallas import tpu_sc as plsc

NUM_GROUP = 26
NUM_ENV = 100000
HIDDEN_DIM = 32
BATCH = 16384

_INFO = plsc.get_sparse_core_info()
_NC, _NS, _L = _INFO.num_cores, _INFO.num_subcores, _INFO.num_lanes
_NW = _NC * _NS
_BPW = BATCH // _NW
_CHUNK = 128
_NCHUNK = _BPW // _CHUNK
_IDX_PER_CHUNK = _CHUNK * NUM_GROUP
_ISTRIDE = 128
_NSTREAM = _IDX_PER_CHUNK // _ISTRIDE


def _sc_body(idx_hbm, table_hbm, out_hbm, idx_v, rows_v, out_v, sem):
    wid = lax.axis_index("s") * _NC + lax.axis_index("c")
    for chunk in range(_NCHUNK):
        base = wid * _BPW + chunk * _CHUNK
        cid = wid * _NCHUNK + chunk
        pltpu.sync_copy(idx_hbm.at[cid], idx_v)

        def fire(k, _):
            vec = idx_v[pl.ds(k * _L, _L)]
            pltpu.async_copy(
                table_hbm.at[vec], rows_v.at[pl.ds(k * _L, _L)], sem)
            return 0

        lax.fori_loop(0, _IDX_PER_CHUNK // _L, fire, 0)

        def drain(k, _):
            pltpu.make_async_copy(
                table_hbm.at[pl.ds(0, _L)],
                rows_v.at[pl.ds(k * _L, _L)], sem).wait()
            return 0

        lax.fori_loop(0, _IDX_PER_CHUNK // _L, drain, 0)

        def body(i, _):
            r = i * NUM_GROUP
            a0 = rows_v[r, pl.ds(0, _L)]
            a1 = rows_v[r, pl.ds(_L, _L)]
            for g in range(1, NUM_GROUP):
                a0 = a0 + rows_v[r + g, pl.ds(0, _L)]
                a1 = a1 + rows_v[r + g, pl.ds(_L, _L)]
            out_v[i, pl.ds(0, _L)] = a0
            out_v[i, pl.ds(_L, _L)] = a1
            return 0

        lax.fori_loop(0, _CHUNK, body, 0)
        pltpu.sync_copy(out_v, out_hbm.at[pl.ds(base, _CHUNK)])


def _lookup_sum(idx_flat, table_flat):
    mesh = plsc.VectorSubcoreMesh(core_axis_name="c", subcore_axis_name="s")
    kern = functools.partial(
        pl.kernel,
        mesh=mesh,
        out_type=jax.ShapeDtypeStruct((BATCH, HIDDEN_DIM), jnp.float32),
        scratch_types=[
            pltpu.VMEM((_IDX_PER_CHUNK,), jnp.int32),
            pltpu.VMEM((_IDX_PER_CHUNK, HIDDEN_DIM), jnp.float32),
            pltpu.VMEM((_CHUNK, HIDDEN_DIM), jnp.float32),
            pltpu.SemaphoreType.DMA,
        ],
        compiler_params=pltpu.CompilerParams(use_tc_tiling_on_sc=False),
    )(_sc_body)
    return kern(idx_flat, table_flat)


def kernel(envs, tables):
    offs = jnp.arange(NUM_GROUP, dtype=jnp.int32) * NUM_ENV
    idx_flat = (envs.astype(jnp.int32) + offs[None, :]).reshape(
        _NW * _NCHUNK, _IDX_PER_CHUNK)
    table_flat = tables.reshape(NUM_GROUP * NUM_ENV, HIDDEN_DIM)
    return _lookup_sum(idx_flat, table_flat)


# speedup vs baseline: 1.0060x; 1.0004x over previous
import functools

import jax
import jax.numpy as jnp
from jax import lax
from jax.experimental import pallas as pl
from jax.experimental.pallas import tpu as pltpu
from jax.experimental.pallas import tpu_sc as plsc

NUM_GROUP = 26
NUM_ENV = 100000
HIDDEN_DIM = 32
BATCH = 16384

_INFO = plsc.get_sparse_core_info()
_NC, _NS, _L = _INFO.num_cores, _INFO.num_subcores, _INFO.num_lanes
_NW = _NC * _NS
_BPW = BATCH // _NW
_CHUNK = 64
_NCHUNK = _BPW // _CHUNK
_IDX_PER_CHUNK = _CHUNK * NUM_GROUP
_NVEC = _IDX_PER_CHUNK // _L


def _sc_body(idx_hbm, table_hbm, out_hbm,
             idx_v0, idx_v1, rows_v0, rows_v1, out_v, sem0, sem1):
    wid = lax.axis_index("s") * _NC + lax.axis_index("c")
    idx_bufs = (idx_v0, idx_v1)
    row_bufs = (rows_v0, rows_v1)
    sems = (sem0, sem1)

    def fetch(chunk, slot):
        idx_v, rows_v, sem = idx_bufs[slot], row_bufs[slot], sems[slot]
        pltpu.sync_copy(idx_hbm.at[wid * _NCHUNK + chunk], idx_v)

        def fire(k, _):
            vec = idx_v[pl.ds(k * _L, _L)]
            pltpu.async_copy(
                table_hbm.at[vec], rows_v.at[pl.ds(k * _L, _L)], sem)
            return 0

        lax.fori_loop(0, _NVEC, fire, 0)

    def drain(slot):
        rows_v, sem = row_bufs[slot], sems[slot]

        def one(k, _):
            pltpu.make_async_copy(
                table_hbm.at[pl.ds(0, _L)],
                rows_v.at[pl.ds(k * _L, _L)], sem).wait()
            return 0

        lax.fori_loop(0, _NVEC, one, 0)

    def reduce_store(chunk, slot):
        rows_v = row_bufs[slot]

        def body(i, _):
            r = i * NUM_GROUP
            a0 = rows_v[r, pl.ds(0, _L)]
            a1 = rows_v[r, pl.ds(_L, _L)]
            for g in range(1, NUM_GROUP):
                a0 = a0 + rows_v[r + g, pl.ds(0, _L)]
                a1 = a1 + rows_v[r + g, pl.ds(_L, _L)]
            out_v[i, pl.ds(0, _L)] = a0
            out_v[i, pl.ds(_L, _L)] = a1
            return 0

        lax.fori_loop(0, _CHUNK, body, 0)
        base = wid * _BPW + chunk * _CHUNK
        pltpu.sync_copy(out_v, out_hbm.at[pl.ds(base, _CHUNK)])

    fetch(0, 0)
    for chunk in range(_NCHUNK):
        if chunk + 1 < _NCHUNK:
            fetch(chunk + 1, (chunk + 1) % 2)
        drain(chunk % 2)
        reduce_store(chunk, chunk % 2)


def _lookup_sum(idx_flat, table_flat):
    mesh = plsc.VectorSubcoreMesh(core_axis_name="c", subcore_axis_name="s")
    kern = functools.partial(
        pl.kernel,
        mesh=mesh,
        out_type=jax.ShapeDtypeStruct((BATCH, HIDDEN_DIM), jnp.float32),
        scratch_types=[
            pltpu.VMEM((_IDX_PER_CHUNK,), jnp.int32),
            pltpu.VMEM((_IDX_PER_CHUNK,), jnp.int32),
            pltpu.VMEM((_IDX_PER_CHUNK, HIDDEN_DIM), jnp.float32),
            pltpu.VMEM((_IDX_PER_CHUNK, HIDDEN_DIM), jnp.float32),
            pltpu.VMEM((_CHUNK, HIDDEN_DIM), jnp.float32),
            pltpu.SemaphoreType.DMA,
            pltpu.SemaphoreType.DMA,
        ],
        compiler_params=pltpu.CompilerParams(use_tc_tiling_on_sc=False),
    )(_sc_body)
    return kern(idx_flat, table_flat)


def kernel(envs, tables):
    offs = jnp.arange(NUM_GROUP, dtype=jnp.int32) * NUM_ENV
    idx_flat = (envs.astype(jnp.int32) + offs[None, :]).reshape(
        _NW * _NCHUNK, _IDX_PER_CHUNK)
    table_flat = tables.reshape(NUM_GROUP * NUM_ENV, HIDDEN_DIM)
    return _lookup_sum(idx_flat, table_flat)


# speedup vs baseline: 1.0108x; 1.0048x over previous
import functools

import jax
import jax.numpy as jnp
from jax import lax
from jax.experimental import pallas as pl
from jax.experimental.pallas import tpu as pltpu
from jax.experimental.pallas import tpu_sc as plsc

NUM_GROUP = 26
NUM_ENV = 100000
HIDDEN_DIM = 32
BATCH = 16384

_INFO = plsc.get_sparse_core_info()
_NC, _NS, _L = _INFO.num_cores, _INFO.num_subcores, _INFO.num_lanes
_NW = _NC * _NS
_BPW = BATCH // _NW
_CHUNK = 64
_NCHUNK = _BPW // _CHUNK
_IDX_PER_W = _BPW * NUM_GROUP
_IDX_PER_CHUNK = _CHUNK * NUM_GROUP
_NVEC = _IDX_PER_CHUNK // _L


def _sc_body(idx_hbm, table_hbm, out_hbm,
             idx_v, rows_v0, rows_v1, out_v, sem0, sem1):
    wid = lax.axis_index("s") * _NC + lax.axis_index("c")
    row_bufs = (rows_v0, rows_v1)
    sems = (sem0, sem1)
    pltpu.sync_copy(idx_hbm.at[wid], idx_v)

    def fetch(chunk, slot):
        rows_v, sem = row_bufs[slot], sems[slot]

        def fire(k, _):
            vec = idx_v[pl.ds(chunk * _IDX_PER_CHUNK + k * _L, _L)]
            pltpu.async_copy(
                table_hbm.at[vec], rows_v.at[pl.ds(k * _L, _L)], sem)
            return 0

        lax.fori_loop(0, _NVEC, fire, 0)

    def drain(slot):
        rows_v, sem = row_bufs[slot], sems[slot]

        def one(k, _):
            pltpu.make_async_copy(
                table_hbm.at[pl.ds(0, _L)],
                rows_v.at[pl.ds(k * _L, _L)], sem).wait()
            return 0

        lax.fori_loop(0, _NVEC, one, 0)

    def reduce_store(chunk, slot):
        rows_v = row_bufs[slot]

        def body(i, _):
            r = i * NUM_GROUP
            a0 = rows_v[r, pl.ds(0, _L)]
            a1 = rows_v[r, pl.ds(_L, _L)]
            for g in range(1, NUM_GROUP):
                a0 = a0 + rows_v[r + g, pl.ds(0, _L)]
                a1 = a1 + rows_v[r + g, pl.ds(_L, _L)]
            out_v[i, pl.ds(0, _L)] = a0
            out_v[i, pl.ds(_L, _L)] = a1
            return 0

        lax.fori_loop(0, _CHUNK, body, 0)
        base = wid * _BPW + chunk * _CHUNK
        pltpu.sync_copy(out_v, out_hbm.at[pl.ds(base, _CHUNK)])

    fetch(0, 0)
    for chunk in range(_NCHUNK):
        if chunk + 1 < _NCHUNK:
            fetch(chunk + 1, (chunk + 1) % 2)
        drain(chunk % 2)
        reduce_store(chunk, chunk % 2)


def _lookup_sum(idx_flat, table_flat):
    mesh = plsc.VectorSubcoreMesh(core_axis_name="c", subcore_axis_name="s")
    kern = functools.partial(
        pl.kernel,
        mesh=mesh,
        out_type=jax.ShapeDtypeStruct((BATCH, HIDDEN_DIM), jnp.float32),
        scratch_types=[
            pltpu.VMEM((_IDX_PER_W,), jnp.int32),
            pltpu.VMEM((_IDX_PER_CHUNK, HIDDEN_DIM), jnp.float32),
            pltpu.VMEM((_IDX_PER_CHUNK, HIDDEN_DIM), jnp.float32),
            pltpu.VMEM((_CHUNK, HIDDEN_DIM), jnp.float32),
            pltpu.SemaphoreType.DMA,
            pltpu.SemaphoreType.DMA,
        ],
        compiler_params=pltpu.CompilerParams(use_tc_tiling_on_sc=False),
    )(_sc_body)
    return kern(idx_flat, table_flat)


def kernel(envs, tables):
    offs = jnp.arange(NUM_GROUP, dtype=jnp.int32) * NUM_ENV
    idx_flat = (envs.astype(jnp.int32) + offs[None, :]).reshape(
        _NW, _IDX_PER_W)
    table_flat = tables.reshape(NUM_GROUP * NUM_ENV, HIDDEN_DIM)
    return _lookup_sum(idx_flat, table_flat)
